# baseline (speedup 1.0000x reference)
import jax
import jax.numpy as jnp
from jax import lax
from jax.experimental import pallas as pl
from jax.experimental.pallas import tpu as pltpu
from jax.experimental.pallas import tpu_sc as plsc

N_CORES = 2
N_SUBCORES = 16
N_WORKERS = N_CORES * N_SUBCORES
LANES = 16


def _gather_body(main_hbm, tail_hbm, idx_hbm, out_hbm,
                 idx_v, idxc_v, rows_v, tail_v, sem_g, sem_tl):
    m = main_hbm.shape[0]
    b_per_w = idx_v.shape[0]
    wid = lax.axis_index("s") * N_CORES + lax.axis_index("c")
    base = wid * b_per_w
    ctl = pltpu.async_copy(tail_hbm, tail_v, sem_tl)
    pltpu.sync_copy(idx_hbm.at[pl.ds(base, b_per_w)], idx_v)

    def clamp(i, _):
        o = i * LANES
        idxc_v[pl.ds(o, LANES)] = jnp.minimum(idx_v[pl.ds(o, LANES)], m - 1)
        return _

    lax.fori_loop(0, b_per_w // LANES, clamp, None)

    cg = pltpu.async_copy(main_hbm.at[idxc_v], rows_v, sem_g)
    ctl.wait()
    cg.wait()

    def patch(i, _):
        o = i * LANES
        vi = idx_v[pl.ds(o, LANES)]
        cond = vi >= m
        ti = jnp.maximum(vi - m, 0)
        vt = plsc.load_gather(tail_v, [ti])
        rows_v[pl.ds(o, LANES)] = jnp.where(cond, vt, rows_v[pl.ds(o, LANES)])
        return _

    lax.fori_loop(0, b_per_w // LANES, patch, None)

    pltpu.sync_copy(rows_v, out_hbm.at[pl.ds(base, b_per_w)])


def _split_flat(a, m):
    return a[:m].reshape(-1), a[m:].reshape(-1)


def _make_gather(n_pad, b_per_w, m, n_tail, name):
    mesh = plsc.VectorSubcoreMesh(
        core_axis_name="c", subcore_axis_name="s",
        num_cores=N_CORES, num_subcores=N_SUBCORES)
    return pl.kernel(
        _gather_body,
        out_type=jax.ShapeDtypeStruct((n_pad,), jnp.float32),
        mesh=mesh,
        scratch_types=[
            pltpu.VMEM((b_per_w,), jnp.int32),
            pltpu.VMEM((b_per_w,), jnp.int32),
            pltpu.VMEM((b_per_w,), jnp.float32),
            pltpu.VMEM((n_tail,), jnp.float32),
            pltpu.SemaphoreType.DMA,
            pltpu.SemaphoreType.DMA,
        ],
        name=name,
        compiler_params=pltpu.CompilerParams(needs_layout_passes=False),
    )


def kernel(x, t, idx):
    n_rows = x.shape[0]
    n = idx.shape[0]
    m = (n_rows // 1024) * 1024
    n_tail = n_rows - m
    b_per_w = -(-n // (LANES * N_WORKERS)) * LANES
    n_pad = b_per_w * N_WORKERS
    idx32 = idx.astype(jnp.int32)
    if n_pad != n:
        idx32 = jnp.concatenate(
            [idx32, jnp.zeros((n_pad - n,), dtype=jnp.int32)])

    main_x, tail_x = _split_flat(x, m)
    main_t, tail_t = _split_flat(t, m)

    gx = _make_gather(n_pad, b_per_w, m, n_tail, "holdout_gather_x")
    gt = _make_gather(n_pad, b_per_w, m, n_tail, "holdout_gather_t")
    out_x = gx(main_x, tail_x, idx32)
    out_t = gt(main_t, tail_t, idx32)

    return (out_x[:n].reshape(n, 1), out_t[:n].reshape(n, 1))


# speedup vs baseline: 1.1205x; 1.1205x over previous
import jax
import jax.numpy as jnp
from jax import lax
from jax.experimental import pallas as pl
from jax.experimental.pallas import tpu as pltpu
from jax.experimental.pallas import tpu_sc as plsc

N_CORES = 2
N_SUBCORES = 16
N_WORKERS = N_CORES * N_SUBCORES


def _gather_body(b_full, b_last, table_hbm, idx_hbm, out_hbm,
                 idx_v, rows_v, sem):
    wid = lax.axis_index("s") * N_CORES + lax.axis_index("c")

    @pl.when(wid < N_WORKERS - 1)
    def _full():
        base = wid * b_full
        pltpu.sync_copy(idx_hbm.at[pl.ds(base, b_full)], idx_v)
        pltpu.async_copy(table_hbm.at[idx_v], rows_v, sem).wait()
        pltpu.sync_copy(rows_v, out_hbm.at[pl.ds(base, b_full)])

    if b_last > 0:
        @pl.when(wid == N_WORKERS - 1)
        def _ragged():
            base = (N_WORKERS - 1) * b_full
            idx_s = idx_v.at[pl.ds(0, b_last)]
            rows_s = rows_v.at[pl.ds(0, b_last)]
            pltpu.sync_copy(idx_hbm.at[pl.ds(base, b_last)], idx_s)
            pltpu.async_copy(table_hbm.at[idx_s], rows_s, sem).wait()
            pltpu.sync_copy(rows_s, out_hbm.at[pl.ds(base, b_last)])


def _flatten_padded(a):
    n_rows = a.shape[0]
    n_pad = -(-n_rows // 1024) * 1024
    if n_pad != n_rows:
        a = jnp.pad(a, ((0, n_pad - n_rows), (0, 0)))
    return a.reshape(-1)


def _make_gather(n, b_full, b_last, name):
    import functools
    mesh = plsc.VectorSubcoreMesh(
        core_axis_name="c", subcore_axis_name="s",
        num_cores=N_CORES, num_subcores=N_SUBCORES)
    return pl.kernel(
        functools.partial(_gather_body, b_full, b_last),
        out_type=jax.ShapeDtypeStruct((n,), jnp.float32),
        mesh=mesh,
        scratch_types=[
            pltpu.VMEM((b_full,), jnp.int32),
            pltpu.VMEM((b_full,), jnp.float32),
            pltpu.SemaphoreType.DMA,
        ],
        name=name,
        compiler_params=pltpu.CompilerParams(needs_layout_passes=False),
    )


def kernel(x, t, idx):
    n = idx.shape[0]
    b_full = -(-n // N_WORKERS) if n % (8 * N_WORKERS) == 0 else (
        -(-n // (8 * N_WORKERS)) * 8)
    b_last = n - b_full * (N_WORKERS - 1)
    assert 0 < b_last <= b_full
    idx32 = idx.astype(jnp.int32)

    t_flat = _flatten_padded(t)
    x_flat = _flatten_padded(x)

    out_t = _make_gather(n, b_full, b_last, "holdout_gather_t")(t_flat, idx32)
    out_x = _make_gather(n, b_full, b_last, "holdout_gather_x")(x_flat, idx32)

    return (out_x.reshape(n, 1), out_t.reshape(n, 1))


# speedup vs baseline: 1.1268x; 1.0056x over previous
import jax
import jax.numpy as jnp
from jax import lax
from jax.experimental import pallas as pl
from jax.experimental.pallas import tpu as pltpu
from jax.experimental.pallas import tpu_sc as plsc

N_CORES = 2
N_SUBCORES = 16
N_WORKERS = N_CORES * N_SUBCORES


def _gather_chunked(table_hbm, idx_hbm, out_hbm, idx_v, rows_v,
                    sem_a, sem_b, base, b, h):
    pltpu.sync_copy(idx_hbm.at[pl.ds(base, h)], idx_v.at[pl.ds(0, h)])
    g1 = pltpu.async_copy(
        table_hbm.at[idx_v.at[pl.ds(0, h)]], rows_v.at[pl.ds(0, h)], sem_a)
    pltpu.sync_copy(idx_hbm.at[pl.ds(base + h, b - h)],
                    idx_v.at[pl.ds(h, b - h)])
    g2 = pltpu.async_copy(
        table_hbm.at[idx_v.at[pl.ds(h, b - h)]],
        rows_v.at[pl.ds(h, b - h)], sem_b)
    g1.wait()
    pltpu.sync_copy(rows_v.at[pl.ds(0, h)], out_hbm.at[pl.ds(base, h)])
    g2.wait()
    pltpu.sync_copy(rows_v.at[pl.ds(h, b - h)],
                    out_hbm.at[pl.ds(base + h, b - h)])


def _gather_body(b_full, b_last, table_hbm, idx_hbm, out_hbm,
                 idx_v, rows_v, sem_a, sem_b):
    wid = lax.axis_index("s") * N_CORES + lax.axis_index("c")

    @pl.when(wid < N_WORKERS - 1)
    def _full():
        base = wid * b_full
        _gather_chunked(table_hbm, idx_hbm, out_hbm, idx_v, rows_v,
                        sem_a, sem_b, base, b_full, (b_full // 16) * 8)

    if b_last > 0:
        @pl.when(wid == N_WORKERS - 1)
        def _ragged():
            base = (N_WORKERS - 1) * b_full
            h = (b_last // 16) * 8
            _gather_chunked(table_hbm, idx_hbm, out_hbm, idx_v, rows_v,
                            sem_a, sem_b, base, b_last, h)


def _flatten_padded(a):
    n_rows = a.shape[0]
    n_pad = -(-n_rows // 1024) * 1024
    if n_pad != n_rows:
        a = jnp.pad(a, ((0, n_pad - n_rows), (0, 0)))
    return a.reshape(-1)


def _make_gather(n, b_full, b_last, name):
    import functools
    mesh = plsc.VectorSubcoreMesh(
        core_axis_name="c", subcore_axis_name="s",
        num_cores=N_CORES, num_subcores=N_SUBCORES)
    return pl.kernel(
        functools.partial(_gather_body, b_full, b_last),
        out_type=jax.ShapeDtypeStruct((n,), jnp.float32),
        mesh=mesh,
        scratch_types=[
            pltpu.VMEM((b_full,), jnp.int32),
            pltpu.VMEM((b_full,), jnp.float32),
            pltpu.SemaphoreType.DMA,
            pltpu.SemaphoreType.DMA,
        ],
        name=name,
        compiler_params=pltpu.CompilerParams(needs_layout_passes=False),
    )


def kernel(x, t, idx):
    n = idx.shape[0]
    b_full = -(-n // N_WORKERS) if n % (8 * N_WORKERS) == 0 else (
        -(-n // (8 * N_WORKERS)) * 8)
    b_last = n - b_full * (N_WORKERS - 1)
    assert 0 < b_last <= b_full
    idx32 = idx.astype(jnp.int32)

    t_flat = _flatten_padded(t)
    x_flat = _flatten_padded(x)

    out_t = _make_gather(n, b_full, b_last, "holdout_gather_t")(t_flat, idx32)
    out_x = _make_gather(n, b_full, b_last, "holdout_gather_x")(x_flat, idx32)

    return (out_x.reshape(n, 1), out_t.reshape(n, 1))
